# Initial kernel scaffold; baseline (speedup 1.0000x reference)
import jax
import jax.numpy as jnp
from jax.experimental import pallas as pl


def kernel(feat, pre_u, pre_v, suc_u, suc_v, W_ctr, W_pre, W_suc, gamma_norm, beta_norm, W_ctr2, gamma_ctr2, beta_ctr2):
    raise NotImplementedError("write your pallas kernel here")


# speedup vs baseline: 5.1118x; 5.1118x over previous
import functools

import jax
import jax.numpy as jnp
from jax import lax
from jax.experimental import pallas as pl
from jax.experimental.pallas import tpu as pltpu
from jax.experimental.pallas import tpu_sc as plsc

N = 10000
D = 128
E = 160000
S = 6
L = 4
J = 2 * S

NC = 2
NS = 16
NW = NC * NS
E_TOT = J * E
E_PW = E_TOT // NW
BLK = 120
NBLK = E_PW // BLK
CH = 25
NCH = NBLK // CH
NP = 10240
ROWS_PT = NP // NS

R = 1000
NR = N // R


def _mm_body(x_ref, wctr_ref, wmsg_ref, yctr_ref, ymsg_ref):
    x = x_ref[...]
    dn = (((1,), (1,)), ((), ()))
    yctr_ref[...] = lax.dot_general(x, wctr_ref[...], dn,
                                    preferred_element_type=jnp.float32)
    for j in range(J):
        ymsg_ref[j] = lax.dot_general(x, wmsg_ref[j], dn,
                                      preferred_element_type=jnp.float32)


@jax.jit
def _mm_call(feat, wctr, wmsg):
    return pl.pallas_call(
        _mm_body,
        grid=(NR,),
        in_specs=[
            pl.BlockSpec((R, D), lambda r: (r, 0)),
            pl.BlockSpec((D, D), lambda r: (0, 0)),
            pl.BlockSpec((J, D, D), lambda r: (0, 0, 0)),
        ],
        out_specs=[
            pl.BlockSpec((R, D), lambda r: (r, 0)),
            pl.BlockSpec((J, R, D), lambda r: (0, r, 0)),
        ],
        out_shape=[
            jax.ShapeDtypeStruct((N, D), jnp.float32),
            jax.ShapeDtypeStruct((J, N, D), jnp.float32),
        ],
    )(feat, wctr, wmsg)


def _gn(x, gamma, beta):
    mean = jnp.mean(x, axis=1, keepdims=True)
    var = jnp.mean((x - mean) ** 2, axis=1, keepdims=True)
    return (x - mean) * lax.rsqrt(var + 1e-5) * gamma + beta


def _tail_body(yctr_ref, p_ref, res_ref, gn_ref, bn_ref, w2_ref, g2_ref,
               b2_ref, out_ref):
    temp = yctr_ref[...] + p_ref[0] + p_ref[1]
    h = jnp.maximum(_gn(temp, gn_ref[...], bn_ref[...]), 0.0)
    dn = (((1,), (1,)), ((), ()))
    g = lax.dot_general(h, w2_ref[...], dn, preferred_element_type=jnp.float32)
    g = _gn(g, g2_ref[...], b2_ref[...])
    out_ref[...] = jnp.maximum(g + res_ref[...], 0.0)


@jax.jit
def _tail_call(yctr, p, res, gn, bn, w2, g2, b2):
    return pl.pallas_call(
        _tail_body,
        grid=(NR,),
        in_specs=[
            pl.BlockSpec((R, D), lambda r: (r, 0)),
            pl.BlockSpec((NC, R, D), lambda r: (0, r, 0)),
            pl.BlockSpec((R, D), lambda r: (r, 0)),
            pl.BlockSpec((1, D), lambda r: (0, 0)),
            pl.BlockSpec((1, D), lambda r: (0, 0)),
            pl.BlockSpec((D, D), lambda r: (0, 0)),
            pl.BlockSpec((1, D), lambda r: (0, 0)),
            pl.BlockSpec((1, D), lambda r: (0, 0)),
        ],
        out_specs=pl.BlockSpec((R, D), lambda r: (r, 0)),
        out_shape=jax.ShapeDtypeStruct((N, D), jnp.float32),
    )(yctr, p, res, gn, bn, w2, g2, b2)


def _sc_body(ymsg_hbm, gidx_hbm, sidx_hbm, zeros_hbm, out_hbm,
             gvm, svm, rows, acc, gsem):
    c = lax.axis_index("c")
    s = lax.axis_index("s")
    w = c * NS + s

    pltpu.sync_copy(zeros_hbm, acc.at[pl.ds(s * ROWS_PT, ROWS_PT)])
    plsc.subcore_barrier()

    def chunk(ci, carry):
        pltpu.sync_copy(gidx_hbm.at[w, ci], gvm)
        pltpu.sync_copy(sidx_hbm.at[w, ci], svm)

        def blk(b, carry2):
            pltpu.async_copy(ymsg_hbm.at[gvm.at[b]], rows, gsem).wait()
            pltpu.sync_copy(rows, acc.at[svm.at[b]], add=True)
            return carry2

        return lax.fori_loop(0, CH, blk, carry)

    lax.fori_loop(0, NCH, chunk, 0)

    plsc.subcore_barrier()
    pltpu.sync_copy(acc.at[pl.ds(s * ROWS_PT, ROWS_PT)],
                    out_hbm.at[c, pl.ds(s * ROWS_PT, ROWS_PT)])


@functools.cache
def _get_sc_call():
    return pl.kernel(
        _sc_body,
        out_type=jax.ShapeDtypeStruct((NC, NP, D), jnp.float32),
        mesh=plsc.VectorSubcoreMesh(core_axis_name="c", subcore_axis_name="s",
                                    num_cores=NC, num_subcores=NS),
        scratch_types=[
            pltpu.VMEM((CH, BLK), jnp.int32),
            pltpu.VMEM((CH, BLK), jnp.int32),
            pltpu.VMEM((BLK, D), jnp.float32),
            pltpu.VMEM_SHARED((NP, D), jnp.float32),
            pltpu.SemaphoreType.DMA,
        ],
    )


def kernel(feat, pre_u, pre_v, suc_u, suc_v, W_ctr, W_pre, W_suc,
           gamma_norm, beta_norm, W_ctr2, gamma_ctr2, beta_ctr2):
    v_all = jnp.concatenate([pre_v, suc_v], axis=0)
    u_all = jnp.concatenate([pre_u, suc_u], axis=0)
    offs = (jnp.arange(J, dtype=jnp.int32) * N)[:, None]
    gidx = (v_all + offs).reshape(NW, NCH, CH, BLK)
    sidx = u_all.reshape(NW, NCH, CH, BLK)
    zeros = jnp.zeros((ROWS_PT, D), jnp.float32)

    wmsg = jnp.concatenate([W_pre, W_suc], axis=1)
    gn = gamma_norm.reshape(L, 1, D)
    bn = beta_norm.reshape(L, 1, D)
    g2 = gamma_ctr2.reshape(L, 1, D)
    b2 = beta_ctr2.reshape(L, 1, D)

    cur = feat
    for i in range(L):
        yctr, ymsg = _mm_call(cur, W_ctr[i], wmsg[i])
        p = _get_sc_call()(ymsg.reshape(J * N, D), gidx, sidx, zeros)
        cur = _tail_call(yctr, p, cur, gn[i], bn[i], W_ctr2[i], g2[i], b2[i])
    return cur
